# baseline (speedup 1.0000x reference)
import jax
import jax.numpy as jnp
from jax import lax
from jax.experimental import pallas as pl
from jax.experimental.pallas import tpu as pltpu
from jax.experimental.pallas import tpu_sc as plsc

B = 16384
D = 64
NC = 2
NS = 16
NW = NC * NS
BPW = B // NW
CHUNK = 128
NCHUNK = BPW // CHUNK
K = 16
PASS = BPW // 2


def _sc_stream_body(uidx_hbm, iidx_hbm, ug_hbm, ig_hbm,
                    ug_out, ig_out,
                    uidx_v, iidx_v, buf0, buf1, sem0, sem1):
  wid = lax.axis_index("s") * NC + lax.axis_index("c")
  base = wid * BPW
  pltpu.sync_copy(uidx_hbm.at[pl.ds(base, BPW)], uidx_v)
  pltpu.sync_copy(iidx_hbm.at[pl.ds(base, BPW)], iidx_v)

  def fire(table, idx_v, buf, sem):
    cps = []
    for j in range(NCHUNK):
      cps.append(pltpu.async_copy(
          table.at[idx_v.at[pl.ds(j * CHUNK, CHUNK)]],
          buf.at[pl.ds(j * CHUNK, CHUNK)], sem))
    return cps

  cps0 = fire(ug_hbm, uidx_v, buf0, sem0)
  cps1 = fire(ig_hbm, iidx_v, buf1, sem1)
  for cp in cps0:
    cp.wait()
  pltpu.sync_copy(buf0, ug_out.at[pl.ds(base, BPW)])
  for cp in cps1:
    cp.wait()
  pltpu.sync_copy(buf1, ig_out.at[pl.ds(base, BPW)])


_sc_stream_gather = pl.kernel(
    _sc_stream_body,
    out_type=[jax.ShapeDtypeStruct((B, D), jnp.float32)] * 2,
    mesh=plsc.VectorSubcoreMesh(core_axis_name="c", subcore_axis_name="s"),
    scratch_types=[
        pltpu.VMEM((BPW,), jnp.int32),
        pltpu.VMEM((BPW,), jnp.int32),
        pltpu.VMEM((BPW, D), jnp.float32),
        pltpu.VMEM((BPW, D), jnp.float32),
        pltpu.SemaphoreType.DMA,
        pltpu.SemaphoreType.DMA,
    ],
    compiler_params=pltpu.CompilerParams(use_tc_tiling_on_sc=False),
)


def _sc_rows_body(uidx_hbm, iidx_hbm, um_hbm, im_hbm,
                  um_out, im_out,
                  uidx_v, iidx_v, um_buf, im_buf, sem):
  wid = lax.axis_index("s") * NC + lax.axis_index("c")
  base = wid * BPW
  pltpu.sync_copy(uidx_hbm.at[pl.ds(base, BPW)], uidx_v)
  pltpu.sync_copy(iidx_hbm.at[pl.ds(base, BPW)], iidx_v)

  def make_wave(half):
    def wave(w, _):
      row0 = half * PASS + w * K
      brow0 = w * K
      uvec = uidx_v[pl.ds(row0, K)]
      ivec = iidx_v[pl.ds(row0, K)]
      for j in range(K):
        ru = uvec[j]
        ri = ivec[j]
        pltpu.async_copy(
            um_hbm.at[pl.ds(ru, 1), :], um_buf.at[pl.ds(brow0 + j, 1), :],
            sem)
        pltpu.async_copy(
            im_hbm.at[pl.ds(ri, 1), :], im_buf.at[pl.ds(brow0 + j, 1), :],
            sem)
      return _
    return wave

  for half in range(2):
    lax.fori_loop(0, PASS // K, make_wave(half), 0)
    for buf in (um_buf, im_buf):
      pltpu.make_async_copy(um_hbm.at[pl.ds(0, PASS), :], buf, sem).wait()
    off = base + half * PASS
    pltpu.sync_copy(um_buf, um_out.at[pl.ds(off, PASS)])
    pltpu.sync_copy(im_buf, im_out.at[pl.ds(off, PASS)])


_sc_rows_gather = pl.kernel(
    _sc_rows_body,
    out_type=[jax.ShapeDtypeStruct((B, D), jnp.float32)] * 2,
    mesh=plsc.VectorSubcoreMesh(core_axis_name="c", subcore_axis_name="s"),
    scratch_types=[
        pltpu.VMEM((BPW,), jnp.int32),
        pltpu.VMEM((BPW,), jnp.int32),
        pltpu.VMEM((PASS, D), jnp.float32),
        pltpu.VMEM((PASS, D), jnp.float32),
        pltpu.SemaphoreType.DMA,
    ],
)

BLK = 2048


def _dense_body(ug_ref, ig_ref, um_ref, im_ref, w1t_ref, b1_ref, wp_ref,
                bp_ref, out_ref):
  gmf = ug_ref[...] * ig_ref[...]
  h = jnp.dot(um_ref[...], w1t_ref[:D, :], preferred_element_type=jnp.float32)
  h = h + jnp.dot(im_ref[...], w1t_ref[D:, :],
                  preferred_element_type=jnp.float32)
  h = jnp.maximum(h + b1_ref[...], 0.0)
  pred = jnp.sum(gmf * wp_ref[:, :D], axis=1)
  pred = pred + jnp.sum(h * wp_ref[:, D:], axis=1)
  out_ref[...] = pred + bp_ref[0, 0]


def _dense_call(ug, ig, um, im, w1t, b1_2d, wp, bp_2d):
  grid = (B // BLK,)
  row_spec = pl.BlockSpec((BLK, D), lambda i: (i, 0))
  return pl.pallas_call(
      _dense_body,
      grid=grid,
      in_specs=[
          row_spec, row_spec, row_spec, row_spec,
          pl.BlockSpec((2 * D, D), lambda i: (0, 0)),
          pl.BlockSpec((1, D), lambda i: (0, 0)),
          pl.BlockSpec((1, 2 * D), lambda i: (0, 0)),
          pl.BlockSpec((1, 1), lambda i: (0, 0)),
      ],
      out_specs=pl.BlockSpec((BLK,), lambda i: (i,)),
      out_shape=jax.ShapeDtypeStruct((B,), jnp.float32),
  )(ug, ig, um, im, w1t, b1_2d, wp, bp_2d)


def kernel(user_indices, item_indices, user_gmf_table, item_gmf_table,
           user_mlp_table, item_mlp_table, W1, b1, Wp, bp):
  uidx = user_indices.astype(jnp.int32)
  iidx = item_indices.astype(jnp.int32)
  ug, ig = _sc_stream_gather(uidx, iidx, user_gmf_table, item_gmf_table)
  um, im = _sc_rows_gather(uidx, iidx, user_mlp_table, item_mlp_table)
  w1t = W1.T
  return _dense_call(ug, ig, um, im, w1t, b1.reshape(1, D), Wp,
                     bp.reshape(1, 1))


# speedup vs baseline: 1.2885x; 1.2885x over previous
import jax
import jax.numpy as jnp
from jax import lax
from jax.experimental import pallas as pl
from jax.experimental.pallas import tpu as pltpu
from jax.experimental.pallas import tpu_sc as plsc

B = 16384
D = 64
V = 100000
NC = 2
NS = 16
NW = NC * NS
RPW = 2
IC = 2048


def _sc_body(uidx_hbm, iidx_hbm, ug_hbm, ig_hbm, um_hbm, im_hbm,
             ug_out, ig_out, um_out, im_out,
             rowbuf, ichunk, obuf, sem):
  wid = lax.axis_index("s") * NC + lax.axis_index("c")
  zero16 = jnp.zeros((16,), jnp.int32)

  def gather_rows(tab_hbm, idx_hbm, out_hbm):
    for r in range(RPW):
      j = wid * RPW + r
      pltpu.sync_copy(tab_hbm.at[pl.ds(j, 1), :], rowbuf)

      def chunk_body(c, _):
        pltpu.sync_copy(idx_hbm.at[pl.ds(c * IC, IC)], ichunk)

        def vec_body(v, _):
          for u in range(4):
            off = v * 64 + u * 16
            iv = ichunk[pl.ds(off, 16)]
            g = plsc.load_gather(rowbuf, [zero16, iv])
            obuf[0, pl.ds(c * IC + off, 16)] = g
          return _

        lax.fori_loop(0, IC // 64, vec_body, 0)
        return _

      lax.fori_loop(0, B // IC, chunk_body, 0)
      pltpu.sync_copy(obuf, out_hbm.at[pl.ds(j, 1), :])

  gather_rows(ug_hbm, uidx_hbm, ug_out)
  gather_rows(um_hbm, uidx_hbm, um_out)
  gather_rows(ig_hbm, iidx_hbm, ig_out)
  gather_rows(im_hbm, iidx_hbm, im_out)


_sc_gather = pl.kernel(
    _sc_body,
    out_type=[jax.ShapeDtypeStruct((D, B), jnp.float32)] * 4,
    mesh=plsc.VectorSubcoreMesh(core_axis_name="c", subcore_axis_name="s"),
    scratch_types=[
        pltpu.VMEM((1, V), jnp.float32),
        pltpu.VMEM((IC,), jnp.int32),
        pltpu.VMEM((1, B), jnp.float32),
        pltpu.SemaphoreType.DMA,
    ],
    compiler_params=pltpu.CompilerParams(needs_layout_passes=False),
)

BLKC = 2048


def _dense_body(ug_ref, ig_ref, um_ref, im_ref, w1_ref, b1_ref, wp_ref,
                bp_ref, out_ref):
  gmf = ug_ref[...] * ig_ref[...]
  h = jnp.dot(w1_ref[:, :D], um_ref[...], preferred_element_type=jnp.float32)
  h = h + jnp.dot(w1_ref[:, D:], im_ref[...],
                  preferred_element_type=jnp.float32)
  h = jnp.maximum(h + b1_ref[...], 0.0)
  pred = jnp.dot(wp_ref[:, :D], gmf, preferred_element_type=jnp.float32)
  pred = pred + jnp.dot(wp_ref[:, D:], h, preferred_element_type=jnp.float32)
  out_ref[...] = pred[0, :] + bp_ref[0, 0]


def _dense_call(ug_t, ig_t, um_t, im_t, W1, b1_2d, wp, bp_2d):
  grid = (B // BLKC,)
  col_spec = pl.BlockSpec((D, BLKC), lambda i: (0, i))
  return pl.pallas_call(
      _dense_body,
      grid=grid,
      in_specs=[
          col_spec, col_spec, col_spec, col_spec,
          pl.BlockSpec((D, 2 * D), lambda i: (0, 0)),
          pl.BlockSpec((D, 1), lambda i: (0, 0)),
          pl.BlockSpec((1, 2 * D), lambda i: (0, 0)),
          pl.BlockSpec((1, 1), lambda i: (0, 0)),
      ],
      out_specs=pl.BlockSpec((BLKC,), lambda i: (i,)),
      out_shape=jax.ShapeDtypeStruct((B,), jnp.float32),
  )(ug_t, ig_t, um_t, im_t, W1, b1_2d, wp, bp_2d)


def kernel(user_indices, item_indices, user_gmf_table, item_gmf_table,
           user_mlp_table, item_mlp_table, W1, b1, Wp, bp):
  uidx = user_indices.astype(jnp.int32)
  iidx = item_indices.astype(jnp.int32)
  ug_t, ig_t, um_t, im_t = _sc_gather(
      uidx, iidx, user_gmf_table.T, item_gmf_table.T,
      user_mlp_table.T, item_mlp_table.T)
  return _dense_call(ug_t, ig_t, um_t, im_t, W1, b1.reshape(D, 1), Wp,
                     bp.reshape(1, 1))


# speedup vs baseline: 1.6753x; 1.3002x over previous
import jax
import jax.numpy as jnp
from jax import lax
from jax.experimental import pallas as pl
from jax.experimental.pallas import tpu as pltpu
from jax.experimental.pallas import tpu_sc as plsc

B = 16384
D = 64
V = 100000
NC = 2
NS = 16
NW = NC * NS
RPW = 2
IC = 2048


def _sc_body(uidx_hbm, iidx_hbm, ug_hbm, ig_hbm, um_hbm, im_hbm,
             ug_out, ig_out, um_out, im_out,
             rowbuf, idxbuf, ob0, ob1, sem, osem):
  wid = lax.axis_index("s") * NC + lax.axis_index("c")
  zero16 = jnp.zeros((16,), jnp.int32)
  obufs = (ob0, ob1)
  pending = [None, None]

  def gather_rows(tab_hbm, out_hbm):
    for r in range(RPW):
      j = wid * RPW + r
      pltpu.sync_copy(tab_hbm.at[pl.ds(j, 1), :], rowbuf)
      for c in range(B // IC):
        ob = obufs[c % 2]
        if pending[c % 2] is not None:
          pending[c % 2].wait()

        def vec_body(v, _, c=c, ob=ob):
          for u in range(4):
            off = v * 64 + u * 16
            iv = idxbuf[pl.ds(c * IC + off, 16)]
            g = plsc.load_gather(rowbuf, [zero16, iv])
            ob[0, pl.ds(off, 16)] = g
          return _

        lax.fori_loop(0, IC // 64, vec_body, 0)
        cp = pltpu.async_copy(
            ob, out_hbm.at[pl.ds(j, 1), pl.ds(c * IC, IC)], osem)
        pending[c % 2] = cp

  pltpu.sync_copy(uidx_hbm, idxbuf)
  gather_rows(ug_hbm, ug_out)
  gather_rows(um_hbm, um_out)
  pltpu.sync_copy(iidx_hbm, idxbuf)
  gather_rows(ig_hbm, ig_out)
  gather_rows(im_hbm, im_out)
  for p in pending:
    if p is not None:
      p.wait()


_sc_gather = pl.kernel(
    _sc_body,
    out_type=[jax.ShapeDtypeStruct((D, B), jnp.float32)] * 4,
    mesh=plsc.VectorSubcoreMesh(core_axis_name="c", subcore_axis_name="s"),
    scratch_types=[
        pltpu.VMEM((1, V), jnp.float32),
        pltpu.VMEM((B,), jnp.int32),
        pltpu.VMEM((1, IC), jnp.float32),
        pltpu.VMEM((1, IC), jnp.float32),
        pltpu.SemaphoreType.DMA,
        pltpu.SemaphoreType.DMA,
    ],
    compiler_params=pltpu.CompilerParams(needs_layout_passes=False),
)

BLKC = 2048


def _dense_body(ug_ref, ig_ref, um_ref, im_ref, w1_ref, b1_ref, wp_ref,
                bp_ref, out_ref):
  gmf = ug_ref[...] * ig_ref[...]
  h = jnp.dot(w1_ref[:, :D], um_ref[...], preferred_element_type=jnp.float32)
  h = h + jnp.dot(w1_ref[:, D:], im_ref[...],
                  preferred_element_type=jnp.float32)
  h = jnp.maximum(h + b1_ref[...], 0.0)
  pred = jnp.dot(wp_ref[:, :D], gmf, preferred_element_type=jnp.float32)
  pred = pred + jnp.dot(wp_ref[:, D:], h, preferred_element_type=jnp.float32)
  out_ref[...] = pred[0, :] + bp_ref[0, 0]


def _dense_call(ug_t, ig_t, um_t, im_t, W1, b1_2d, wp, bp_2d):
  grid = (B // BLKC,)
  col_spec = pl.BlockSpec((D, BLKC), lambda i: (0, i))
  return pl.pallas_call(
      _dense_body,
      grid=grid,
      in_specs=[
          col_spec, col_spec, col_spec, col_spec,
          pl.BlockSpec((D, 2 * D), lambda i: (0, 0)),
          pl.BlockSpec((D, 1), lambda i: (0, 0)),
          pl.BlockSpec((1, 2 * D), lambda i: (0, 0)),
          pl.BlockSpec((1, 1), lambda i: (0, 0)),
      ],
      out_specs=pl.BlockSpec((BLKC,), lambda i: (i,)),
      out_shape=jax.ShapeDtypeStruct((B,), jnp.float32),
  )(ug_t, ig_t, um_t, im_t, W1, b1_2d, wp, bp_2d)


def kernel(user_indices, item_indices, user_gmf_table, item_gmf_table,
           user_mlp_table, item_mlp_table, W1, b1, Wp, bp):
  uidx = user_indices.astype(jnp.int32)
  iidx = item_indices.astype(jnp.int32)
  ug_t, ig_t, um_t, im_t = _sc_gather(
      uidx, iidx, user_gmf_table.T, item_gmf_table.T,
      user_mlp_table.T, item_mlp_table.T)
  return _dense_call(ug_t, ig_t, um_t, im_t, W1, b1.reshape(D, 1), Wp,
                     bp.reshape(1, 1))


# speedup vs baseline: 2.0891x; 1.2470x over previous
import jax
import jax.numpy as jnp
from jax import lax
from jax.experimental import pallas as pl
from jax.experimental.pallas import tpu as pltpu
from jax.experimental.pallas import tpu_sc as plsc

B = 16384
D = 64
V = 100000
NC = 2
NS = 16
NW = NC * NS
RPW = 2
IC = 2048


def _sc_body(uidx_hbm, iidx_hbm, ug_hbm, ig_hbm, um_hbm, im_hbm,
             ug_out, ig_out, um_out, im_out,
             rowbuf, idxbuf, ob0, ob1, sem, osem):
  wid = lax.axis_index("s") * NC + lax.axis_index("c")
  zero16 = jnp.zeros((16,), jnp.int32)
  obufs = (ob0, ob1)
  pending = [None, None]

  def gather_rows(tab_hbm, out_hbm):
    for r in range(RPW):
      j = wid * RPW + r
      pltpu.sync_copy(tab_hbm.at[pl.ds(j, 1), :], rowbuf)
      for c in range(B // IC):
        ob = obufs[c % 2]
        if pending[c % 2] is not None:
          pending[c % 2].wait()

        def vec_body(v, _, c=c, ob=ob):
          for u in range(8):
            off = v * 128 + u * 16
            iv = idxbuf[pl.ds(c * IC + off, 16)]
            g = plsc.load_gather(rowbuf, [zero16, iv])
            ob[0, pl.ds(off, 16)] = g
          return _

        lax.fori_loop(0, IC // 128, vec_body, 0)
        cp = pltpu.async_copy(
            ob, out_hbm.at[pl.ds(j, 1), pl.ds(c * IC, IC)], osem)
        pending[c % 2] = cp

  pltpu.sync_copy(uidx_hbm, idxbuf)
  gather_rows(ug_hbm, ug_out)
  gather_rows(um_hbm, um_out)
  pltpu.sync_copy(iidx_hbm, idxbuf)
  gather_rows(ig_hbm, ig_out)
  gather_rows(im_hbm, im_out)
  for p in pending:
    if p is not None:
      p.wait()


_sc_gather = pl.kernel(
    _sc_body,
    out_type=[jax.ShapeDtypeStruct((D, B), jnp.float32)] * 4,
    mesh=plsc.VectorSubcoreMesh(core_axis_name="c", subcore_axis_name="s"),
    scratch_types=[
        pltpu.VMEM((1, V), jnp.float32),
        pltpu.VMEM((B,), jnp.int32),
        pltpu.VMEM((1, IC), jnp.float32),
        pltpu.VMEM((1, IC), jnp.float32),
        pltpu.SemaphoreType.DMA,
        pltpu.SemaphoreType.DMA,
    ],
    compiler_params=pltpu.CompilerParams(needs_layout_passes=False),
)

BLKC = 2048


def _dense_body(ug_ref, ig_ref, um_ref, im_ref, w1_ref, b1_ref, wp_ref,
                bp_ref, out_ref):
  gmf = ug_ref[...] * ig_ref[...]
  h = jnp.dot(w1_ref[:, :D], um_ref[...], preferred_element_type=jnp.float32)
  h = h + jnp.dot(w1_ref[:, D:], im_ref[...],
                  preferred_element_type=jnp.float32)
  h = jnp.maximum(h + b1_ref[...], 0.0)
  pred = jnp.dot(wp_ref[:, :D], gmf, preferred_element_type=jnp.float32)
  pred = pred + jnp.dot(wp_ref[:, D:], h, preferred_element_type=jnp.float32)
  out_ref[...] = pred[0, :] + bp_ref[0, 0]


def _dense_call(ug_t, ig_t, um_t, im_t, W1, b1_2d, wp, bp_2d):
  grid = (B // BLKC,)
  col_spec = pl.BlockSpec((D, BLKC), lambda i: (0, i))
  return pl.pallas_call(
      _dense_body,
      grid=grid,
      in_specs=[
          col_spec, col_spec, col_spec, col_spec,
          pl.BlockSpec((D, 2 * D), lambda i: (0, 0)),
          pl.BlockSpec((D, 1), lambda i: (0, 0)),
          pl.BlockSpec((1, 2 * D), lambda i: (0, 0)),
          pl.BlockSpec((1, 1), lambda i: (0, 0)),
      ],
      out_specs=pl.BlockSpec((BLKC,), lambda i: (i,)),
      out_shape=jax.ShapeDtypeStruct((B,), jnp.float32),
  )(ug_t, ig_t, um_t, im_t, W1, b1_2d, wp, bp_2d)


def kernel(user_indices, item_indices, user_gmf_table, item_gmf_table,
           user_mlp_table, item_mlp_table, W1, b1, Wp, bp):
  uidx = user_indices.astype(jnp.int32)
  iidx = item_indices.astype(jnp.int32)
  ug_t, ig_t, um_t, im_t = _sc_gather(
      uidx, iidx, user_gmf_table.T, item_gmf_table.T,
      user_mlp_table.T, item_mlp_table.T)
  return _dense_call(ug_t, ig_t, um_t, im_t, W1, b1.reshape(D, 1), Wp,
                     bp.reshape(1, 1))
